# Initial kernel scaffold; baseline (speedup 1.0000x reference)
import functools

import jax
import jax.numpy as jnp
from jax import lax
from jax.experimental import pallas as pl
from jax.experimental.pallas import tpu as pltpu
from jax.experimental.pallas import tpu_sc as plsc

N = 10000
D = 128
E = 320000
NC, NS, L = 2, 16, 16
NW = NC * NS
EPW = E // NW
C = 128
NFULL = EPW // C
TAIL = EPW - NFULL * C
NPAD = 10240
RPT = NPAD // NS

_MESH = plsc.VectorSubcoreMesh(core_axis_name="c", subcore_axis_name="s")


def _worker_id():
    return lax.axis_index("s") * NC + lax.axis_index("c")


def _deg_body(dst_hbm, deg_part, ones_v, onest_v, idx_v, idxt_v, gath_v,
              out_v, acc_sh):
    cid = lax.axis_index("c")
    sid = lax.axis_index("s")
    wid = _worker_id()

    one16 = jnp.full((L,), 1.0, jnp.float32)
    zero16 = jnp.zeros((L,), jnp.float32)

    def fill(r, _):
        ones_v[r, :] = one16
        gath_v[r, :] = zero16
        return 0
    lax.fori_loop(0, C, fill, 0)
    for r in range(TAIL):
        onest_v[r, :] = one16

    for z in range(RPT // C):
        pltpu.sync_copy(gath_v, acc_sh.at[pl.ds(sid * RPT + z * C, C), :])
    plsc.subcore_barrier()

    ebase = wid * EPW

    def chunk(c, _):
        pltpu.sync_copy(dst_hbm.at[pl.ds(ebase + c * C, C)], idx_v)
        pltpu.sync_copy(ones_v, acc_sh.at[idx_v], add=True)
        return 0
    lax.fori_loop(0, NFULL, chunk, 0)

    pltpu.sync_copy(dst_hbm.at[pl.ds(ebase + NFULL * C, TAIL)], idxt_v)
    pltpu.sync_copy(onest_v, acc_sh.at[idxt_v], add=True)
    plsc.subcore_barrier()

    col0 = jnp.zeros((L,), jnp.int32)
    for z in range(RPT // C):
        pltpu.sync_copy(acc_sh.at[pl.ds(sid * RPT + z * C, C), :], gath_v)

        def ext(i, _):
            rows = lax.iota(jnp.int32, L) + i * L
            out_v[pl.ds(z * C + i * L, L)] = plsc.load_gather(
                gath_v, [rows, col0])
            return 0
        lax.fori_loop(0, C // L, ext, 0)
    pltpu.sync_copy(out_v, deg_part.at[cid, pl.ds(sid * RPT, RPT)])


_deg_call = pl.kernel(
    _deg_body,
    out_type=jax.ShapeDtypeStruct((NC, NPAD), jnp.float32),
    mesh=_MESH,
    scratch_types=[
        pltpu.VMEM((C, L), jnp.float32),
        pltpu.VMEM((TAIL, L), jnp.float32),
        pltpu.VMEM((C,), jnp.int32),
        pltpu.VMEM((TAIL,), jnp.int32),
        pltpu.VMEM((C, L), jnp.float32),
        pltpu.VMEM((RPT,), jnp.float32),
        pltpu.VMEM_SHARED((NPAD, L), jnp.float32),
    ],
)


def _agg_body(g_hbm, src_hbm, dst_hbm, part, srcb, dstb, srct, dstt,
              rows_v, rowst_v, acc_sh, sem):
    cid = lax.axis_index("c")
    sid = lax.axis_index("s")
    wid = _worker_id()

    zero16 = jnp.zeros((L,), jnp.float32)

    def fill(r, _):
        for k in range(D // L):
            rows_v[r, pl.ds(k * L, L)] = zero16
        return 0
    lax.fori_loop(0, C, fill, 0)

    for z in range(RPT // C):
        pltpu.sync_copy(rows_v, acc_sh.at[pl.ds(sid * RPT + z * C, C), :])
    plsc.subcore_barrier()

    ebase = wid * EPW

    def chunk(c, _):
        base = ebase + c * C
        pltpu.sync_copy(src_hbm.at[pl.ds(base, C)], srcb)
        pltpu.sync_copy(dst_hbm.at[pl.ds(base, C)], dstb)
        pltpu.async_copy(g_hbm.at[srcb], rows_v, sem).wait()
        pltpu.sync_copy(rows_v, acc_sh.at[dstb], add=True)
        return 0
    lax.fori_loop(0, NFULL, chunk, 0)

    tbase = ebase + NFULL * C
    pltpu.sync_copy(src_hbm.at[pl.ds(tbase, TAIL)], srct)
    pltpu.sync_copy(dst_hbm.at[pl.ds(tbase, TAIL)], dstt)
    pltpu.async_copy(g_hbm.at[srct], rowst_v, sem).wait()
    pltpu.sync_copy(rowst_v, acc_sh.at[dstt], add=True)
    plsc.subcore_barrier()

    for z in range(RPT // C):
        sl = pl.ds(sid * RPT + z * C, C)
        pltpu.sync_copy(acc_sh.at[sl, :], rows_v)
        pltpu.sync_copy(rows_v, part.at[cid, sl, :])


_agg_call = pl.kernel(
    _agg_body,
    out_type=jax.ShapeDtypeStruct((NC, NPAD, D), jnp.float32),
    mesh=_MESH,
    scratch_types=[
        pltpu.VMEM((C,), jnp.int32),
        pltpu.VMEM((C,), jnp.int32),
        pltpu.VMEM((TAIL,), jnp.int32),
        pltpu.VMEM((TAIL,), jnp.int32),
        pltpu.VMEM((C, D), jnp.float32),
        pltpu.VMEM((TAIL, D), jnp.float32),
        pltpu.VMEM_SHARED((NPAD, D), jnp.float32),
        pltpu.SemaphoreType.DMA,
    ],
)


def _dinv_body(degp_ref, o_ref):
    deg = degp_ref[0, :] + degp_ref[1, :] + 1.0
    o_ref[0, :] = lax.rsqrt(deg)


_dinv_call = pl.pallas_call(
    _dinv_body,
    out_shape=jax.ShapeDtypeStruct((1, NPAD), jnp.float32),
)

_RB = 2000
_GRID = N // _RB


def _mm1_body(d_ref, x_ref, w_ref, o_ref):
    h = jnp.dot(x_ref[...], w_ref[...], preferred_element_type=jnp.float32)
    o_ref[...] = d_ref[...] * h


_mm1_call = pl.pallas_call(
    _mm1_body,
    grid=(_GRID,),
    in_specs=[
        pl.BlockSpec((_RB, 1), lambda i: (i, 0)),
        pl.BlockSpec((_RB, D), lambda i: (i, 0)),
        pl.BlockSpec((D, D), lambda i: (0, 0)),
    ],
    out_specs=pl.BlockSpec((_RB, D), lambda i: (i, 0)),
    out_shape=jax.ShapeDtypeStruct((N, D), jnp.float32),
)


def _mid_body(p_ref, g_ref, d_ref, b_ref, w_ref, o_ref):
    agg = p_ref[0] + p_ref[1]
    z = jnp.maximum(d_ref[...] * (agg + g_ref[...]) + b_ref[...], 0.0)
    o_ref[...] = d_ref[...] * jnp.dot(
        z, w_ref[...], preferred_element_type=jnp.float32)


_mid_call = pl.pallas_call(
    _mid_body,
    grid=(_GRID,),
    in_specs=[
        pl.BlockSpec((NC, _RB, D), lambda i: (0, i, 0)),
        pl.BlockSpec((_RB, D), lambda i: (i, 0)),
        pl.BlockSpec((_RB, 1), lambda i: (i, 0)),
        pl.BlockSpec((1, D), lambda i: (0, 0)),
        pl.BlockSpec((D, D), lambda i: (0, 0)),
    ],
    out_specs=pl.BlockSpec((_RB, D), lambda i: (i, 0)),
    out_shape=jax.ShapeDtypeStruct((N, D), jnp.float32),
)


def _fin_body(q_ref, g_ref, d_ref, b_ref, o_ref):
    agg = q_ref[0] + q_ref[1]
    o_ref[...] = d_ref[...] * (agg + g_ref[...]) + b_ref[...]


_fin_call = pl.pallas_call(
    _fin_body,
    grid=(_GRID,),
    in_specs=[
        pl.BlockSpec((NC, _RB, D), lambda i: (0, i, 0)),
        pl.BlockSpec((_RB, D), lambda i: (i, 0)),
        pl.BlockSpec((_RB, 1), lambda i: (i, 0)),
        pl.BlockSpec((1, D), lambda i: (0, 0)),
    ],
    out_specs=pl.BlockSpec((_RB, D), lambda i: (i, 0)),
    out_shape=jax.ShapeDtypeStruct((N, D), jnp.float32),
)


@jax.jit
def kernel(x, edge_index, W1, b1, W2, b2):
    src = edge_index[0].astype(jnp.int32)
    dst = edge_index[1].astype(jnp.int32)

    deg_part = _deg_call(dst)
    dinv = _dinv_call(deg_part)
    dcol = dinv.reshape(NPAD, 1)[:N]

    b1r = b1.reshape(1, D)
    b2r = b2.reshape(1, D)

    g1 = _mm1_call(dcol, x, W1)
    p = _agg_call(g1, src, dst)
    g2 = _mid_call(p[:, :N], g1, dcol, b1r, W2)
    q = _agg_call(g2, src, dst)
    return _fin_call(q[:, :N], g2, dcol, b2r)


# speedup vs baseline: 15.9737x; 15.9737x over previous
import functools

import jax
import jax.numpy as jnp
from jax import lax
from jax.experimental import pallas as pl
from jax.experimental.pallas import tpu as pltpu
from jax.experimental.pallas import tpu_sc as plsc

N = 10000
D = 128
E = 320000
NC, NS, L = 2, 16, 16
NW = NC * NS
EPW = E // NW
C = 128
NFULL = EPW // C
TAIL = EPW - NFULL * C
NPAD = 10240
RPT = NPAD // NS

_MESH = plsc.VectorSubcoreMesh(core_axis_name="c", subcore_axis_name="s")


def _worker_id():
    return lax.axis_index("s") * NC + lax.axis_index("c")


def _deg_body(dst_hbm, deg_part, ones_v, onest_v, idx_v, idxt_v, gath_v,
              acc_sh):
    cid = lax.axis_index("c")
    sid = lax.axis_index("s")
    wid = _worker_id()

    one16 = jnp.full((L,), 1.0, jnp.float32)
    zero16 = jnp.zeros((L,), jnp.float32)

    def fill(r, _):
        ones_v[r, :] = one16
        gath_v[r, :] = zero16
        return 0
    lax.fori_loop(0, C, fill, 0)
    for r in range(TAIL):
        onest_v[r, :] = one16

    for z in range(RPT // C):
        pltpu.sync_copy(gath_v, acc_sh.at[pl.ds(sid * RPT + z * C, C), :])
    plsc.subcore_barrier()

    ebase = wid * EPW

    def chunk(c, _):
        pltpu.sync_copy(dst_hbm.at[pl.ds(ebase + c * C, C)], idx_v)
        pltpu.sync_copy(ones_v, acc_sh.at[idx_v], add=True)
        return 0
    lax.fori_loop(0, NFULL, chunk, 0)

    pltpu.sync_copy(dst_hbm.at[pl.ds(ebase + NFULL * C, TAIL)], idxt_v)
    pltpu.sync_copy(onest_v, acc_sh.at[idxt_v], add=True)
    plsc.subcore_barrier()

    for z in range(RPT // C):
        sl = pl.ds(sid * RPT + z * C, C)
        pltpu.sync_copy(acc_sh.at[sl, :], gath_v)
        pltpu.sync_copy(gath_v, deg_part.at[cid, sl, :])


_deg_call = pl.kernel(
    _deg_body,
    out_type=jax.ShapeDtypeStruct((NC, NPAD, L), jnp.float32),
    mesh=_MESH,
    scratch_types=[
        pltpu.VMEM((C, L), jnp.float32),
        pltpu.VMEM((TAIL, L), jnp.float32),
        pltpu.VMEM((C,), jnp.int32),
        pltpu.VMEM((TAIL,), jnp.int32),
        pltpu.VMEM((C, L), jnp.float32),
        pltpu.VMEM_SHARED((NPAD, L), jnp.float32),
    ],
)


def _agg_body(g_hbm, src_hbm, dst_hbm, part, srcb, dstb, srct, dstt,
              rows_v, rowst_v, acc_sh, sem):
    cid = lax.axis_index("c")
    sid = lax.axis_index("s")
    wid = _worker_id()

    zero16 = jnp.zeros((L,), jnp.float32)

    def fill(r, _):
        for k in range(D // L):
            rows_v[r, pl.ds(k * L, L)] = zero16
        return 0
    lax.fori_loop(0, C, fill, 0)

    for z in range(RPT // C):
        pltpu.sync_copy(rows_v, acc_sh.at[pl.ds(sid * RPT + z * C, C), :])
    plsc.subcore_barrier()

    ebase = wid * EPW

    def chunk(c, _):
        base = ebase + c * C
        pltpu.sync_copy(src_hbm.at[pl.ds(base, C)], srcb)
        pltpu.sync_copy(dst_hbm.at[pl.ds(base, C)], dstb)
        pltpu.async_copy(g_hbm.at[srcb], rows_v, sem).wait()
        pltpu.sync_copy(rows_v, acc_sh.at[dstb], add=True)
        return 0
    lax.fori_loop(0, NFULL, chunk, 0)

    tbase = ebase + NFULL * C
    pltpu.sync_copy(src_hbm.at[pl.ds(tbase, TAIL)], srct)
    pltpu.sync_copy(dst_hbm.at[pl.ds(tbase, TAIL)], dstt)
    pltpu.async_copy(g_hbm.at[srct], rowst_v, sem).wait()
    pltpu.sync_copy(rowst_v, acc_sh.at[dstt], add=True)
    plsc.subcore_barrier()

    for z in range(RPT // C):
        sl = pl.ds(sid * RPT + z * C, C)
        pltpu.sync_copy(acc_sh.at[sl, :], rows_v)
        pltpu.sync_copy(rows_v, part.at[cid, sl, :])


_agg_call = pl.kernel(
    _agg_body,
    out_type=jax.ShapeDtypeStruct((NC, NPAD, D), jnp.float32),
    mesh=_MESH,
    scratch_types=[
        pltpu.VMEM((C,), jnp.int32),
        pltpu.VMEM((C,), jnp.int32),
        pltpu.VMEM((TAIL,), jnp.int32),
        pltpu.VMEM((TAIL,), jnp.int32),
        pltpu.VMEM((C, D), jnp.float32),
        pltpu.VMEM((TAIL, D), jnp.float32),
        pltpu.VMEM_SHARED((NPAD, D), jnp.float32),
        pltpu.SemaphoreType.DMA,
    ],
)


def _dinv_body(degp_ref, o_ref):
    deg = degp_ref[0, :, 0:1] + degp_ref[1, :, 0:1] + 1.0
    o_ref[...] = lax.rsqrt(deg)


_dinv_call = pl.pallas_call(
    _dinv_body,
    out_shape=jax.ShapeDtypeStruct((NPAD, 1), jnp.float32),
)

_RB = 2000
_GRID = N // _RB


def _mm1_body(d_ref, x_ref, w_ref, o_ref):
    h = jnp.dot(x_ref[...], w_ref[...], preferred_element_type=jnp.float32)
    o_ref[...] = d_ref[...] * h


_mm1_call = pl.pallas_call(
    _mm1_body,
    grid=(_GRID,),
    in_specs=[
        pl.BlockSpec((_RB, 1), lambda i: (i, 0)),
        pl.BlockSpec((_RB, D), lambda i: (i, 0)),
        pl.BlockSpec((D, D), lambda i: (0, 0)),
    ],
    out_specs=pl.BlockSpec((_RB, D), lambda i: (i, 0)),
    out_shape=jax.ShapeDtypeStruct((N, D), jnp.float32),
)


def _mid_body(p_ref, g_ref, d_ref, b_ref, w_ref, o_ref):
    agg = p_ref[0] + p_ref[1]
    z = jnp.maximum(d_ref[...] * (agg + g_ref[...]) + b_ref[...], 0.0)
    o_ref[...] = d_ref[...] * jnp.dot(
        z, w_ref[...], preferred_element_type=jnp.float32)


_mid_call = pl.pallas_call(
    _mid_body,
    grid=(_GRID,),
    in_specs=[
        pl.BlockSpec((NC, _RB, D), lambda i: (0, i, 0)),
        pl.BlockSpec((_RB, D), lambda i: (i, 0)),
        pl.BlockSpec((_RB, 1), lambda i: (i, 0)),
        pl.BlockSpec((1, D), lambda i: (0, 0)),
        pl.BlockSpec((D, D), lambda i: (0, 0)),
    ],
    out_specs=pl.BlockSpec((_RB, D), lambda i: (i, 0)),
    out_shape=jax.ShapeDtypeStruct((N, D), jnp.float32),
)


def _fin_body(q_ref, g_ref, d_ref, b_ref, o_ref):
    agg = q_ref[0] + q_ref[1]
    o_ref[...] = d_ref[...] * (agg + g_ref[...]) + b_ref[...]


_fin_call = pl.pallas_call(
    _fin_body,
    grid=(_GRID,),
    in_specs=[
        pl.BlockSpec((NC, _RB, D), lambda i: (0, i, 0)),
        pl.BlockSpec((_RB, D), lambda i: (i, 0)),
        pl.BlockSpec((_RB, 1), lambda i: (i, 0)),
        pl.BlockSpec((1, D), lambda i: (0, 0)),
    ],
    out_specs=pl.BlockSpec((_RB, D), lambda i: (i, 0)),
    out_shape=jax.ShapeDtypeStruct((N, D), jnp.float32),
)


@jax.jit
def kernel(x, edge_index, W1, b1, W2, b2):
    src = edge_index[0].astype(jnp.int32)
    dst = edge_index[1].astype(jnp.int32)

    deg_part = _deg_call(dst)
    dcol = _dinv_call(deg_part)[:N]

    b1r = b1.reshape(1, D)
    b2r = b2.reshape(1, D)

    g1 = _mm1_call(dcol, x, W1)
    p = _agg_call(g1, src, dst)
    g2 = _mid_call(p[:, :N], g1, dcol, b1r, W2)
    q = _agg_call(g2, src, dst)
    return _fin_call(q[:, :N], g2, dcol, b2r)
